# initial kernel scaffold (unmeasured)
import functools

import jax
import jax.numpy as jnp
from jax import lax
from jax.experimental import pallas as pl
from jax.experimental.pallas import tpu as pltpu

N_DEV = 8
B, S, C = 4, 2048, 1024
TAPS = 4


def _body(x_ref, x3_ref, k_ref, out_ref, halo_ref, send_sem, recv_sem):
    b = pl.program_id(0)
    my_i = lax.axis_index("i")

    @pl.when(b == 0)
    def _comm():
        @pl.when(my_i == 0)
        def _():
            halo_ref[...] = jnp.zeros_like(halo_ref)

        @pl.when(my_i < N_DEV - 1)
        def _():
            send = pltpu.make_async_remote_copy(
                src_ref=x3_ref,
                dst_ref=halo_ref,
                send_sem=send_sem,
                recv_sem=recv_sem,
                device_id=(my_i + 1,),
                device_id_type=pl.DeviceIdType.MESH,
            )
            send.start()
            send.wait_send()

        @pl.when(my_i > 0)
        def _():
            recv = pltpu.make_async_remote_copy(
                src_ref=x3_ref,
                dst_ref=halo_ref,
                send_sem=send_sem,
                recv_sem=recv_sem,
                device_id=(my_i - 1,),
                device_id_type=pl.DeviceIdType.MESH,
            )
            recv.wait_recv()

    xb = x_ref[0, :, :]
    h = halo_ref[pl.ds(b, 1), :, :][0]
    full = jnp.concatenate([h, xb], axis=0)
    acc = full[TAPS - 1 : TAPS - 1 + S] * k_ref[TAPS - 1]
    for t in range(TAPS - 1):
        acc += full[t : t + S] * k_ref[t]
    out_ref[0, :, :] = acc / (1.0 + jnp.exp(-acc))


def kernel(x, k):
    x3 = lax.slice_in_dim(x, S - (TAPS - 1), S, axis=1)

    return pl.pallas_call(
        _body,
        grid=(B,),
        in_specs=[
            pl.BlockSpec((1, S, C), lambda b: (b, 0, 0)),
            pl.BlockSpec((B, TAPS - 1, C), lambda b: (0, 0, 0)),
            pl.BlockSpec((TAPS, C), lambda b: (0, 0)),
        ],
        out_specs=pl.BlockSpec((1, S, C), lambda b: (b, 0, 0)),
        out_shape=jax.ShapeDtypeStruct((B, S, C), jnp.float32),
        scratch_shapes=[
            pltpu.VMEM((B, TAPS - 1, C), jnp.float32),
            pltpu.SemaphoreType.DMA,
            pltpu.SemaphoreType.DMA,
        ],
        compiler_params=pltpu.CompilerParams(collective_id=0),
    )(x, x3, k)


# baseline (device time: 71509 ns/iter reference)
import functools

import jax
import jax.numpy as jnp
from jax import lax
from jax.experimental import pallas as pl
from jax.experimental.pallas import tpu as pltpu

N_DEV = 8
B, S, C = 4, 2048, 1024
TAPS = 4


def _body(x_ref, x3_ref, k_ref, out_ref, halo_ref, send_sem, recv_sem):
    b = pl.program_id(0)
    my_i = lax.axis_index("i")

    @pl.when(b == 0)
    def _comm():
        @pl.when(my_i == 0)
        def _():
            halo_ref[...] = jnp.zeros_like(halo_ref)

        @pl.when(my_i < N_DEV - 1)
        def _():
            send = pltpu.make_async_remote_copy(
                src_ref=x3_ref,
                dst_ref=halo_ref,
                send_sem=send_sem,
                recv_sem=recv_sem,
                device_id=(my_i + 1,),
                device_id_type=pl.DeviceIdType.MESH,
            )
            send.start()
            send.wait_send()

        @pl.when(my_i > 0)
        def _():
            recv = pltpu.make_async_remote_copy(
                src_ref=x3_ref,
                dst_ref=halo_ref,
                send_sem=send_sem,
                recv_sem=recv_sem,
                device_id=(my_i - 1,),
                device_id_type=pl.DeviceIdType.MESH,
            )
            recv.wait_recv()

    xb = x_ref[0, :, :]
    h = halo_ref[pl.ds(b, 1), :, :][0]
    full = jnp.concatenate([h, xb], axis=0)
    acc = full[TAPS - 1 : TAPS - 1 + S] * k_ref[TAPS - 1]
    for t in range(TAPS - 1):
        acc += full[t : t + S] * k_ref[t]
    out_ref[0, :, :] = acc / (1.0 + jnp.exp(-acc))


def kernel(x, k):
    x3 = lax.slice_in_dim(x, S - (TAPS - 1), S, axis=1)

    return pl.pallas_call(
        _body,
        grid=(B,),
        in_specs=[
            pl.BlockSpec((1, S, C), lambda b: (b, 0, 0)),
            pl.BlockSpec((B, TAPS - 1, C), lambda b: (0, 0, 0)),
            pl.BlockSpec((TAPS, C), lambda b: (0, 0)),
        ],
        out_specs=pl.BlockSpec((1, S, C), lambda b: (b, 0, 0)),
        out_shape=jax.ShapeDtypeStruct((B, S, C), jnp.float32),
        scratch_shapes=[
            pltpu.VMEM((B, TAPS - 1, C), jnp.float32),
            pltpu.SemaphoreType.DMA,
            pltpu.SemaphoreType.DMA,
        ],
        compiler_params=pltpu.CompilerParams(vmem_limit_bytes=48 * 1024 * 1024),
    )(x, x3, k)
